# Initial kernel scaffold; baseline (speedup 1.0000x reference)
import jax
import jax.numpy as jnp
from jax.experimental import pallas as pl


def kernel(input_ids, table, embed_scale):
    raise NotImplementedError("write your pallas kernel here")


# speedup vs baseline: 4.8823x; 4.8823x over previous
import functools
import jax
import jax.numpy as jnp
from jax import lax
from jax.experimental import pallas as pl
from jax.experimental.pallas import tpu as pltpu
from jax.experimental.pallas import tpu_sc as plsc

NC, NS, L = 2, 16, 16
NW = NC * NS
D = 128
G = 128


def _make_kernel(B):
    assert B % (NW * G) == 0
    groups_per_w = B // (NW * G)
    rows_per_w = B // NW

    mesh = plsc.VectorSubcoreMesh(
        core_axis_name="c", subcore_axis_name="s", num_cores=NC, num_subcores=NS
    )

    @functools.partial(
        pl.kernel,
        out_type=jax.ShapeDtypeStruct((B, D), jnp.float32),
        mesh=mesh,
        scratch_types=[
            pltpu.VMEM((groups_per_w, G), jnp.int32),
            pltpu.VMEM((G, D), jnp.float32),
            pltpu.VMEM((L,), jnp.float32),
            pltpu.SemaphoreType.DMA,
        ],
    )
    def k(ids_hbm, table_hbm, scale_hbm, out_hbm, idx_v, rows_v, scale_v, sem):
        wid = lax.axis_index("s") * NC + lax.axis_index("c")
        pltpu.sync_copy(ids_hbm.at[wid], idx_v)
        pltpu.sync_copy(scale_hbm, scale_v)
        s_vec = scale_v[...]

        def group_body(j, carry):
            pltpu.async_copy(table_hbm.at[idx_v.at[j]], rows_v, sem).wait()

            def row_body(r, c2):
                for cg in range(D // L):
                    sl = pl.ds(cg * L, L)
                    rows_v[r, sl] = rows_v[r, sl] * s_vec
                return c2

            lax.fori_loop(0, G, row_body, 0)
            pltpu.sync_copy(
                rows_v, out_hbm.at[pl.ds(wid * rows_per_w + j * G, G)]
            )
            return carry

        lax.fori_loop(0, groups_per_w, group_body, 0)

    return k


def kernel(input_ids, table, embed_scale):
    B, S = input_ids.shape
    n = B * S
    ids2d = input_ids.reshape(NW, n // (NW * G), G)
    scale16 = jnp.broadcast_to(embed_scale.astype(jnp.float32), (L,))
    out = _make_kernel(n)(ids2d, table, scale16)
    return out.reshape(B, S, D)


# speedup vs baseline: 8.0300x; 1.6447x over previous
import functools
import jax
import jax.numpy as jnp
from jax import lax
from jax.experimental import pallas as pl
from jax.experimental.pallas import tpu as pltpu
from jax.experimental.pallas import tpu_sc as plsc

NC, NS, L = 2, 16, 16
NW = NC * NS
D = 128
G = 128
NBUF = 2


def _make_kernel(B):
    assert B % (NW * G) == 0
    ngroups = B // (NW * G)
    assert ngroups % NBUF == 0
    rows_per_w = B // NW

    mesh = plsc.VectorSubcoreMesh(
        core_axis_name="c", subcore_axis_name="s", num_cores=NC, num_subcores=NS
    )

    @functools.partial(
        pl.kernel,
        out_type=jax.ShapeDtypeStruct((B, D), jnp.float32),
        mesh=mesh,
        scratch_types=[
            pltpu.VMEM((ngroups, G), jnp.int32),
            pltpu.VMEM((NBUF, G, D), jnp.float32),
            pltpu.VMEM((NBUF, G, D), jnp.float32),
            pltpu.VMEM((L,), jnp.float32),
            pltpu.SemaphoreType.DMA,
            pltpu.SemaphoreType.DMA,
            pltpu.SemaphoreType.DMA,
            pltpu.SemaphoreType.DMA,
        ],
    )
    def k(ids_hbm, table_hbm, scale_hbm, out_hbm,
          idx_v, gbuf, wbuf, scale_v, sg0, sg1, sw0, sw1):
        wid = lax.axis_index("s") * NC + lax.axis_index("c")
        base = wid * rows_per_w
        sgs = (sg0, sg1)
        sws = (sw0, sw1)

        pltpu.sync_copy(ids_hbm.at[wid], idx_v)
        pltpu.sync_copy(scale_hbm, scale_v)
        s_vec = scale_v[...]

        def start_gather(j, b):
            pltpu.make_async_copy(
                table_hbm.at[idx_v.at[j]], gbuf.at[b], sgs[b]
            ).start()

        def wait_gather(j, b):
            pltpu.make_async_copy(
                table_hbm.at[idx_v.at[j]], gbuf.at[b], sgs[b]
            ).wait()

        def start_write(j, b):
            pltpu.make_async_copy(
                wbuf.at[b], out_hbm.at[pl.ds(base + j * G, G)], sws[b]
            ).start()

        def wait_write(j, b):
            pltpu.make_async_copy(
                wbuf.at[b], out_hbm.at[pl.ds(base + j * G, G)], sws[b]
            ).wait()

        def scale_group(b):
            def row_body(r, c2):
                for u in range(2):
                    for cg in range(D // L):
                        sl = pl.ds(cg * L, L)
                        wbuf[b, 2 * r + u, sl] = gbuf[b, 2 * r + u, sl] * s_vec
                return c2

            lax.fori_loop(0, G // 2, row_body, 0)

        for b in range(NBUF):
            start_gather(b, b)

        def t_body(t, carry):
            for b in range(NBUF):
                j = t * NBUF + b
                wait_gather(j, b)

                @pl.when(t >= 1)
                def _():
                    wait_write(j - NBUF, b)

                scale_group(b)

                @pl.when(t < ngroups // NBUF - 1)
                def _():
                    start_gather(j + NBUF, b)

                start_write(j, b)
            return carry

        lax.fori_loop(0, ngroups // NBUF, t_body, 0)

        for b in range(NBUF):
            wait_write(ngroups - NBUF + b, b)

    return k


def kernel(input_ids, table, embed_scale):
    B, S = input_ids.shape
    n = B * S
    ids3d = input_ids.reshape(NW, n // (NW * G), G)
    scale16 = jnp.broadcast_to(embed_scale.astype(jnp.float32), (L,))
    out = _make_kernel(n)(ids3d, table, scale16)
    return out.reshape(B, S, D)


# speedup vs baseline: 8.0926x; 1.0078x over previous
import functools
import jax
import jax.numpy as jnp
from jax import lax
from jax.experimental import pallas as pl
from jax.experimental.pallas import tpu as pltpu
from jax.experimental.pallas import tpu_sc as plsc

NC, NS, L = 2, 16, 16
NW = NC * NS
D = 128
G = 128
NBG = 5
NBW = 2
STEP = 10


def _make_kernel(B):
    assert B % (NW * G) == 0
    ngroups = B // (NW * G)
    assert ngroups % STEP == 0
    nsteps = ngroups // STEP
    rows_per_w = B // NW

    mesh = plsc.VectorSubcoreMesh(
        core_axis_name="c", subcore_axis_name="s", num_cores=NC, num_subcores=NS
    )

    @functools.partial(
        pl.kernel,
        out_type=jax.ShapeDtypeStruct((B, D), jnp.float32),
        mesh=mesh,
        scratch_types=[
            pltpu.VMEM((ngroups, G), jnp.int32),
            pltpu.VMEM((NBG, G, D), jnp.float32),
            pltpu.VMEM((NBW, G, D), jnp.float32),
            pltpu.VMEM((L,), jnp.float32),
            [pltpu.SemaphoreType.DMA] * NBG,
            [pltpu.SemaphoreType.DMA] * NBW,
        ],
    )
    def k(ids_hbm, table_hbm, scale_hbm, out_hbm,
          idx_v, gbuf, wbuf, scale_v, sgs, sws):
        wid = lax.axis_index("s") * NC + lax.axis_index("c")
        base = wid * rows_per_w

        pltpu.sync_copy(ids_hbm.at[wid], idx_v)
        pltpu.sync_copy(scale_hbm, scale_v)
        s_vec = scale_v[...]

        def start_gather(j, bg):
            pltpu.make_async_copy(
                table_hbm.at[idx_v.at[j]], gbuf.at[bg], sgs[bg]
            ).start()

        def wait_gather(j, bg):
            pltpu.make_async_copy(
                table_hbm.at[idx_v.at[j]], gbuf.at[bg], sgs[bg]
            ).wait()

        def start_write(j, bw):
            pltpu.make_async_copy(
                wbuf.at[bw], out_hbm.at[pl.ds(base + j * G, G)], sws[bw]
            ).start()

        def wait_write(j, bw):
            pltpu.make_async_copy(
                wbuf.at[bw], out_hbm.at[pl.ds(base + j * G, G)], sws[bw]
            ).wait()

        def scale_group(bg, bw):
            def row_body(r, c2):
                for u in range(4):
                    for cg in range(D // L):
                        sl = pl.ds(cg * L, L)
                        wbuf[bw, 4 * r + u, sl] = gbuf[bg, 4 * r + u, sl] * s_vec
                return c2

            lax.fori_loop(0, G // 4, row_body, 0)

        for b in range(NBG):
            start_gather(b, b)

        def t_body(t, carry):
            for b in range(STEP):
                j = t * STEP + b
                bg = b % NBG
                bw = b % NBW
                wait_gather(j, bg)

                if b >= NBW:
                    wait_write(j - NBW, bw)
                else:
                    @pl.when(t >= 1)
                    def _():
                        wait_write(j - NBW, bw)

                scale_group(bg, bw)

                if b < STEP - NBG:
                    start_gather(j + NBG, bg)
                else:
                    @pl.when(t < nsteps - 1)
                    def _():
                        start_gather(j + NBG, bg)

                start_write(j, bw)
            return carry

        lax.fori_loop(0, nsteps, t_body, 0)

        for b in range(NBW):
            wait_write(ngroups - NBW + b, b)

    return k


def kernel(input_ids, table, embed_scale):
    B, S = input_ids.shape
    n = B * S
    ids3d = input_ids.reshape(NW, n // (NW * G), G)
    scale16 = jnp.broadcast_to(embed_scale.astype(jnp.float32), (L,))
    out = _make_kernel(n)(ids3d, table, scale16)
    return out.reshape(B, S, D)
